# baseline (speedup 1.0000x reference)
import jax
import jax.numpy as jnp
from jax.experimental import pallas as pl

_GRID = 5


def _copy_body(n_ref, ei_ref, e_ref, u_ref, b_ref,
               no_ref, eio_ref, eo_ref, uo_ref, bo_ref):
    no_ref[...] = n_ref[...]
    eio_ref[...] = ei_ref[...]
    eo_ref[...] = e_ref[...]
    uo_ref[...] = u_ref[...]
    bo_ref[...] = b_ref[...]


def kernel(nodes, edge_index, edges=None, u=None, batch=None):
    if batch is None:
        batch = jnp.zeros((nodes.shape[0],), dtype=jnp.int32)

    n_rows, d_feat = nodes.shape
    n_edges, d_edge = edges.shape

    ei2 = edge_index.reshape(5000, 128)
    e2 = edges.reshape(40000, 128)
    b2 = batch.reshape(80, 125)
    g = _GRID
    nb, eib, eb, bb = n_rows // g, 5000 // g, 40000 // g, 80 // g

    out = pl.pallas_call(
        _copy_body,
        grid=(g,),
        in_specs=[
            pl.BlockSpec((nb, d_feat), lambda i: (i, 0)),
            pl.BlockSpec((eib, 128), lambda i: (i, 0)),
            pl.BlockSpec((eb, 128), lambda i: (i, 0)),
            pl.BlockSpec((1, d_feat), lambda i: (0, 0)),
            pl.BlockSpec((bb, 125), lambda i: (i, 0)),
        ],
        out_specs=[
            pl.BlockSpec((nb, d_feat), lambda i: (i, 0)),
            pl.BlockSpec((eib, 128), lambda i: (i, 0)),
            pl.BlockSpec((eb, 128), lambda i: (i, 0)),
            pl.BlockSpec((1, d_feat), lambda i: (0, 0)),
            pl.BlockSpec((bb, 125), lambda i: (i, 0)),
        ],
        out_shape=[
            jax.ShapeDtypeStruct(nodes.shape, nodes.dtype),
            jax.ShapeDtypeStruct(ei2.shape, edge_index.dtype),
            jax.ShapeDtypeStruct(e2.shape, edges.dtype),
            jax.ShapeDtypeStruct(u.shape, u.dtype),
            jax.ShapeDtypeStruct(b2.shape, batch.dtype),
        ],
    )(nodes, ei2, e2, u, b2)

    nodes_o, ei_o, edges_o, u_o, b_o = out
    return (nodes_o, ei_o.reshape(edge_index.shape),
            edges_o.reshape(edges.shape), u_o, b_o.reshape(batch.shape))
